# baseline (speedup 1.0000x reference)
import functools

import jax
import jax.numpy as jnp
from jax import lax
from jax.experimental import pallas as pl
from jax.experimental.pallas import tpu as pltpu
from jax.experimental.pallas import tpu_sc as plsc

USERS = 50000
ITEMS = 50000
N = 100000
N_PAD = 100096
EMB = 32
HALF = 16
E = 1600000
SUB = 128
K = 6
CHUNK = K * SUB
NCH = E // CHUNK
E_MAIN = NCH * CHUNK
TAIL = E - E_MAIN
NCORE = 2
NSUB = 16
TRIPS = -(-NCH // NSUB)
T2 = (TRIPS + 2) // 2
ROWS_PER_TILE = N_PAD // NSUB
BATCH = 4096
B2 = 2 * BATCH
BPT = B2 // NSUB
FSUB = 256

_PARAMS = pltpu.CompilerParams(use_tc_tiling_on_sc=False,
                               needs_layout_passes=False)

_MESH = plsc.VectorSubcoreMesh(
    core_axis_name="c", subcore_axis_name="s", num_cores=NCORE,
    num_subcores=NSUB)


def _gcn_body(ego0, cols3, rows3, vals3, cols_t, rows_t, vals_t, nid3,
              out, t1, t2, t3,
              cbA, cbB, rbA, rbB, vbA, vbB, gA, gB, acc,
              sem_cbA, sem_cbB, sem_rvA, sem_rvB,
              sem_gA, sem_gB, sem_sA, sem_sB, sem_z):
    c = lax.axis_index("c")
    tid = lax.axis_index("s")
    row0 = tid * ROWS_PER_TILE

    bufA = (cbA, rbA, gA, sem_cbA, sem_rvA, sem_gA, sem_sA, vbA)
    bufB = (cbB, rbB, gB, sem_cbB, sem_rvB, sem_gB, sem_sB, vbB)

    def fire_cb(i, buf):
        pltpu.async_copy(cols3.at[i], buf[0], buf[3])

    def wait_cb(buf):
        pltpu.make_async_copy(cols3.at[0], buf[0], buf[3]).wait()

    def fire_rv(i, buf):
        pltpu.async_copy(rows3.at[i], buf[1], buf[4])
        pltpu.async_copy(vals3.at[i], buf[7], buf[4])

    def wait_rv(buf):
        pltpu.make_async_copy(rows3.at[0], buf[1], buf[4]).wait()
        pltpu.make_async_copy(vals3.at[0], buf[7], buf[4]).wait()

    def spmm_phase(src, dst, flat_src=False):
        def adjust_cols(cb):
            if not flat_src:
                return

            def abody(gi, carry):
                v = cb[gi // 8, pl.ds((gi % 8) * 16, 16)]
                cb[gi // 8, pl.ds((gi % 8) * 16, 16)] = v * 2 + c
                return carry
            lax.fori_loop(0, CHUNK // 16, abody, None, unroll=4)

        def gather_src(cb, j):
            if flat_src:
                return src.at[cb.at[j]]
            return src.at[c].at[cb.at[j]]

        def fire_gathers(buf):
            cb, g, sem_g = buf[0], buf[2], buf[5]
            for j in range(K):
                pltpu.async_copy(gather_src(cb, j),
                                 g.at[pl.ds(j * SUB, SUB)], sem_g)

        def wait_gathers(buf):
            cb, g, sem_g = buf[0], buf[2], buf[5]
            for j in range(K):
                pltpu.make_async_copy(gather_src(cb, j),
                                      g.at[pl.ds(j * SUB, SUB)], sem_g).wait()

        def fire_scatters(buf):
            rv, g, sem_s = buf[1], buf[2], buf[6]
            for j in range(K):
                pltpu.async_copy(g.at[pl.ds(j * SUB, SUB)],
                                 acc.at[rv.at[j]], sem_s, add=True)

        def wait_scatters(buf):
            rv, g, sem_s = buf[1], buf[2], buf[6]
            for j in range(K):
                pltpu.make_async_copy(g.at[pl.ds(j * SUB, SUB)],
                                      acc.at[rv.at[j]], sem_s).wait()

        def scale(buf):
            vb, g = buf[7], buf[2]

            def body(gi, carry):
                v = vb[gi // 8, pl.ds((gi % 8) * 16, 16)]
                base = gi * 16
                for l in range(16):
                    g[base + l, :] = g[base + l, :] * v[l]
                return carry
            lax.fori_loop(0, CHUNK // 16, body, None, unroll=2)

        fire_cb(tid, bufA)
        fire_rv(tid, bufA)
        fire_cb(NSUB + tid, bufB)
        wait_cb(bufA)
        adjust_cols(bufA[0])
        fire_gathers(bufA)

        def zbody(i, carry):
            gB[i, :] = jnp.zeros((HALF,), jnp.float32)
            return carry
        lax.fori_loop(0, CHUNK, zbody, None, unroll=8)
        zds = []
        for q in range(ROWS_PER_TILE // CHUNK):
            zds.append(pltpu.async_copy(
                gB, acc.at[pl.ds(row0 + q * CHUNK, CHUNK)], sem_z))
        tail = ROWS_PER_TILE % CHUNK
        if tail:
            zds.append(pltpu.async_copy(
                gB.at[pl.ds(0, tail)],
                acc.at[pl.ds(row0 + ROWS_PER_TILE - tail, tail)], sem_z))
        for d in zds:
            d.wait()
        plsc.subcore_barrier()

        def half(t, cur, nxt):
            i_cur = t * NSUB + tid
            i_prev = i_cur - NSUB
            i_next = i_cur + NSUB
            i_next2 = i_cur + 2 * NSUB

            @pl.when(i_cur < NCH)
            def _():
                wait_gathers(cur)

            @pl.when((t >= 1) & (i_prev < NCH))
            def _():
                wait_scatters(nxt)

            @pl.when(i_next < NCH)
            def _():
                fire_rv(i_next, nxt)

            @pl.when(i_next2 < NCH)
            def _():
                fire_cb(i_next2, cur)

            @pl.when(i_next < NCH)
            def _():
                wait_cb(nxt)
                adjust_cols(nxt[0])
                fire_gathers(nxt)

            @pl.when(i_cur < NCH)
            def _():
                wait_rv(cur)
                scale(cur)
                fire_scatters(cur)

        def pipe(t2_, carry):
            half(2 * t2_, bufA, bufB)
            half(2 * t2_ + 1, bufB, bufA)
            return carry
        lax.fori_loop(0, T2, pipe, None)

        @pl.when(tid == NSUB - 1)
        def _():
            d1 = pltpu.async_copy(cols_t, cbA.at[pl.ds(0, 2)], sem_cbA)
            d2 = pltpu.async_copy(rows_t, rbA.at[pl.ds(0, 2)], sem_rvA)
            d3 = pltpu.async_copy(vals_t, vbA.at[pl.ds(0, 2)], sem_rvA)
            d1.wait()
            d2.wait()
            d3.wait()
            if flat_src:
                def tadj(gi, carry):
                    v = cbA[gi // 8, pl.ds((gi % 8) * 16, 16)]
                    cbA[gi // 8, pl.ds((gi % 8) * 16, 16)] = v * 2 + c
                    return carry
                lax.fori_loop(0, TAIL // 16, tadj, None, unroll=4)
            tds = [pltpu.async_copy(gather_src(cbA, j),
                                    gA.at[pl.ds(j * SUB, SUB)], sem_gA)
                   for j in range(TAIL // SUB)]
            for d in tds:
                d.wait()

            def tbody(gi, carry):
                v = vbA[gi // 8, pl.ds((gi % 8) * 16, 16)]
                base = gi * 16
                for l in range(16):
                    gA[base + l, :] = gA[base + l, :] * v[l]
                return carry
            lax.fori_loop(0, TAIL // 16, tbody, None, unroll=2)
            sds = [pltpu.async_copy(gA.at[pl.ds(j * SUB, SUB)],
                                    acc.at[rbA.at[j]], sem_sA, add=True)
                   for j in range(TAIL // SUB)]
            for d in sds:
                d.wait()

        plsc.subcore_barrier()
        pltpu.sync_copy(acc.at[pl.ds(row0, ROWS_PER_TILE)],
                        dst.at[c].at[pl.ds(row0, ROWS_PER_TILE)])
        plsc.subcore_barrier()

    spmm_phase(ego0, t1, flat_src=True)
    spmm_phase(t1, t2)
    spmm_phase(t2, t3)

    for p in range(2):
        pltpu.async_copy(nid3.at[tid * 2 + p], cbA.at[pl.ds(0, 2)],
                         sem_cbA).wait()

        def fadj(gi, carry):
            v = cbA[gi // 8, pl.ds((gi % 8) * 16, 16)]
            cbB[gi // 8, pl.ds((gi % 8) * 16, 16)] = v * 2 + c
            return carry
        lax.fori_loop(0, FSUB // 16, fadj, None, unroll=4)
        descs = []
        for q in range(2):
            descs.append(pltpu.async_copy(
                ego0.at[cbB.at[q]], gA.at[pl.ds(q * SUB, SUB)], sem_gA))
        for li, tbl in enumerate((t1, t2, t3)):
            for q in range(2):
                dgbuf = gA if li < 2 else gB
                doff = (li + 1) * FSUB if li < 2 else 0
                descs.append(pltpu.async_copy(
                    tbl.at[c].at[cbA.at[q]],
                    dgbuf.at[pl.ds(doff + q * SUB, SUB)], sem_gA))
        for d in descs:
            d.wait()

        def mean(e, carry):
            m = (gA[e, :] + gA[FSUB + e, :] + gA[2 * FSUB + e, :]
                 + gB[e, :]) * 0.25
            gB[FSUB + e, :] = m
            return carry
        lax.fori_loop(0, FSUB, mean, None, unroll=8)

        pltpu.sync_copy(
            gB.at[pl.ds(FSUB, FSUB)],
            out.at[c].at[pl.ds(tid * BPT + p * FSUB, FSUB)])


_gcn = functools.partial(
    pl.kernel,
    out_type=jax.ShapeDtypeStruct((NCORE, B2, HALF), jnp.float32),
    mesh=_MESH,
    compiler_params=_PARAMS,
    scratch_types=[
        pltpu.HBM((NCORE, N_PAD, HALF), jnp.float32),
        pltpu.HBM((NCORE, N_PAD, HALF), jnp.float32),
        pltpu.HBM((NCORE, N_PAD, HALF), jnp.float32),
        pltpu.VMEM((K, SUB), jnp.int32),
        pltpu.VMEM((K, SUB), jnp.int32),
        pltpu.VMEM((K, SUB), jnp.int32),
        pltpu.VMEM((K, SUB), jnp.int32),
        pltpu.VMEM((K, SUB), jnp.float32),
        pltpu.VMEM((K, SUB), jnp.float32),
        pltpu.VMEM((CHUNK, HALF), jnp.float32),
        pltpu.VMEM((CHUNK, HALF), jnp.float32),
        pltpu.VMEM_SHARED((N_PAD, HALF), jnp.float32),
        pltpu.SemaphoreType.DMA,
        pltpu.SemaphoreType.DMA,
        pltpu.SemaphoreType.DMA,
        pltpu.SemaphoreType.DMA,
        pltpu.SemaphoreType.DMA,
        pltpu.SemaphoreType.DMA,
        pltpu.SemaphoreType.DMA,
        pltpu.SemaphoreType.DMA,
        pltpu.SemaphoreType.DMA,
    ],
)(_gcn_body)


def kernel(users, items, user_emb, item_emb, adj_rows, adj_cols, adj_vals):
    ego0 = jnp.concatenate(
        [user_emb, item_emb,
         jnp.zeros((N_PAD - N, EMB), jnp.float32)], axis=0)
    ego0 = ego0.reshape(NCORE * N_PAD, HALF)

    cols3 = adj_cols[:E_MAIN].reshape(NCH, K, SUB)
    rows3 = adj_rows[:E_MAIN].reshape(NCH, K, SUB)
    vals3 = adj_vals[:E_MAIN].reshape(NCH, K, SUB)
    cols_t = adj_cols[E_MAIN:].reshape(TAIL // SUB, SUB)
    rows_t = adj_rows[E_MAIN:].reshape(TAIL // SUB, SUB)
    vals_t = adj_vals[E_MAIN:].reshape(TAIL // SUB, SUB)

    nid = jnp.concatenate(
        [users.astype(jnp.int32), items.astype(jnp.int32) + USERS])
    nid3 = nid.reshape(NSUB * 2, 2, SUB)

    outf = _gcn(ego0, cols3, rows3, vals3, cols_t, rows_t, vals_t, nid3)

    o = outf.transpose(1, 0, 2).reshape(B2, EMB)
    return (o[:BATCH], o[BATCH:])


# speedup vs baseline: 1.0099x; 1.0099x over previous
import functools

import jax
import jax.numpy as jnp
from jax import lax
from jax.experimental import pallas as pl
from jax.experimental.pallas import tpu as pltpu
from jax.experimental.pallas import tpu_sc as plsc

USERS = 50000
ITEMS = 50000
N = 100000
N_PAD = 100096
EMB = 32
HALF = 16
E = 1600000
SUB = 128
K = 6
CHUNK = K * SUB
NCH = E // CHUNK
E_MAIN = NCH * CHUNK
TAIL = E - E_MAIN
NCORE = 2
NSUB = 16
TRIPS = -(-NCH // NSUB)
T2 = (TRIPS + 2) // 2
ROWS_PER_TILE = N_PAD // NSUB
BATCH = 4096
B2 = 2 * BATCH
BPT = B2 // NSUB
FSUB = 256

_PARAMS = pltpu.CompilerParams(use_tc_tiling_on_sc=False,
                               needs_layout_passes=False)

_MESH = plsc.VectorSubcoreMesh(
    core_axis_name="c", subcore_axis_name="s", num_cores=NCORE,
    num_subcores=NSUB)


def _gcn_body(ego0, cols3, rows3, vals3, cols_t, rows_t, vals_t, nid3,
              out, t1, t2, t3,
              cbA, cbB, rbA, rbB, vbA, vbB, gA, gB, acc,
              sem_cbA, sem_cbB, sem_rvA, sem_rvB,
              sem_gA, sem_gB, sem_sA, sem_sB, sem_z):
    c = lax.axis_index("c")
    tid = lax.axis_index("s")
    row0 = tid * ROWS_PER_TILE

    bufA = (cbA, rbA, gA, sem_cbA, sem_rvA, sem_gA, sem_sA, vbA)
    bufB = (cbB, rbB, gB, sem_cbB, sem_rvB, sem_gB, sem_sB, vbB)

    def fire_cb(i, buf):
        pltpu.async_copy(cols3.at[i], buf[0], buf[3])

    def wait_cb(buf):
        pltpu.make_async_copy(cols3.at[0], buf[0], buf[3]).wait()

    def fire_rv(i, buf):
        pltpu.async_copy(rows3.at[i], buf[1], buf[4])
        pltpu.async_copy(vals3.at[i], buf[7], buf[4])

    def wait_rv(buf):
        pltpu.make_async_copy(rows3.at[0], buf[1], buf[4]).wait()
        pltpu.make_async_copy(vals3.at[0], buf[7], buf[4]).wait()

    def spmm_phase(src, dst):
        def fire_gathers(buf):
            cb, g, sem_g = buf[0], buf[2], buf[5]
            for j in range(K):
                pltpu.async_copy(src.at[c].at[cb.at[j]],
                                 g.at[pl.ds(j * SUB, SUB)], sem_g)

        def wait_gathers(buf):
            cb, g, sem_g = buf[0], buf[2], buf[5]
            for j in range(K):
                pltpu.make_async_copy(src.at[c].at[cb.at[j]],
                                      g.at[pl.ds(j * SUB, SUB)], sem_g).wait()

        def fire_scatters(buf):
            rv, g, sem_s = buf[1], buf[2], buf[6]
            for j in range(K):
                pltpu.async_copy(g.at[pl.ds(j * SUB, SUB)],
                                 acc.at[rv.at[j]], sem_s, add=True)

        def wait_scatters(buf):
            rv, g, sem_s = buf[1], buf[2], buf[6]
            for j in range(K):
                pltpu.make_async_copy(g.at[pl.ds(j * SUB, SUB)],
                                      acc.at[rv.at[j]], sem_s).wait()

        def scale(buf):
            vb, g = buf[7], buf[2]

            def body(gi, carry):
                v = vb[gi // 8, pl.ds((gi % 8) * 16, 16)]
                base = gi * 16
                for l in range(16):
                    g[base + l, :] = g[base + l, :] * v[l]
                return carry
            lax.fori_loop(0, CHUNK // 16, body, None, unroll=2)

        fire_cb(tid, bufA)
        fire_rv(tid, bufA)
        fire_cb(NSUB + tid, bufB)
        wait_cb(bufA)
        fire_gathers(bufA)

        def zbody(i, carry):
            gB[i, :] = jnp.zeros((HALF,), jnp.float32)
            return carry
        lax.fori_loop(0, CHUNK, zbody, None, unroll=8)
        zds = []
        for q in range(ROWS_PER_TILE // CHUNK):
            zds.append(pltpu.async_copy(
                gB, acc.at[pl.ds(row0 + q * CHUNK, CHUNK)], sem_z))
        tail = ROWS_PER_TILE % CHUNK
        if tail:
            zds.append(pltpu.async_copy(
                gB.at[pl.ds(0, tail)],
                acc.at[pl.ds(row0 + ROWS_PER_TILE - tail, tail)], sem_z))
        for d in zds:
            d.wait()
        plsc.subcore_barrier()

        def half(t, cur, nxt):
            i_cur = t * NSUB + tid
            i_prev = i_cur - NSUB
            i_next = i_cur + NSUB
            i_next2 = i_cur + 2 * NSUB

            @pl.when(i_cur < NCH)
            def _():
                wait_gathers(cur)

            @pl.when((t >= 1) & (i_prev < NCH))
            def _():
                wait_scatters(nxt)

            @pl.when(i_next < NCH)
            def _():
                fire_rv(i_next, nxt)

            @pl.when(i_next2 < NCH)
            def _():
                fire_cb(i_next2, cur)

            @pl.when(i_next < NCH)
            def _():
                wait_cb(nxt)
                fire_gathers(nxt)

            @pl.when(i_cur < NCH)
            def _():
                wait_rv(cur)
                scale(cur)
                fire_scatters(cur)

        def pipe(t2_, carry):
            half(2 * t2_, bufA, bufB)
            half(2 * t2_ + 1, bufB, bufA)
            return carry
        lax.fori_loop(0, T2, pipe, None)

        @pl.when(tid == NSUB - 1)
        def _():
            d1 = pltpu.async_copy(cols_t, cbA.at[pl.ds(0, 2)], sem_cbA)
            d2 = pltpu.async_copy(rows_t, rbA.at[pl.ds(0, 2)], sem_rvA)
            d3 = pltpu.async_copy(vals_t, vbA.at[pl.ds(0, 2)], sem_rvA)
            d1.wait()
            d2.wait()
            d3.wait()
            tds = [pltpu.async_copy(src.at[c].at[cbA.at[j]],
                                    gA.at[pl.ds(j * SUB, SUB)], sem_gA)
                   for j in range(TAIL // SUB)]
            for d in tds:
                d.wait()

            def tbody(gi, carry):
                v = vbA[gi // 8, pl.ds((gi % 8) * 16, 16)]
                base = gi * 16
                for l in range(16):
                    gA[base + l, :] = gA[base + l, :] * v[l]
                return carry
            lax.fori_loop(0, TAIL // 16, tbody, None, unroll=2)
            sds = [pltpu.async_copy(gA.at[pl.ds(j * SUB, SUB)],
                                    acc.at[rbA.at[j]], sem_sA, add=True)
                   for j in range(TAIL // SUB)]
            for d in sds:
                d.wait()

        plsc.subcore_barrier()
        pltpu.sync_copy(acc.at[pl.ds(row0, ROWS_PER_TILE)],
                        dst.at[c].at[pl.ds(row0, ROWS_PER_TILE)])
        plsc.subcore_barrier()

    spmm_phase(ego0, t1)
    spmm_phase(t1, t2)
    spmm_phase(t2, t3)

    for p in range(2):
        pltpu.async_copy(nid3.at[tid * 2 + p], cbA.at[pl.ds(0, 2)],
                         sem_cbA).wait()
        descs = []
        for li, tbl in enumerate((ego0, t1, t2, t3)):
            for q in range(2):
                dgbuf = gA if li < 3 else gB
                doff = li * FSUB if li < 3 else 0
                descs.append(pltpu.async_copy(
                    tbl.at[c].at[cbA.at[q]],
                    dgbuf.at[pl.ds(doff + q * SUB, SUB)], sem_gA))
        for d in descs:
            d.wait()

        def mean(e, carry):
            m = (gA[e, :] + gA[FSUB + e, :] + gA[2 * FSUB + e, :]
                 + gB[e, :]) * 0.25
            gB[FSUB + e, :] = m
            return carry
        lax.fori_loop(0, FSUB, mean, None, unroll=8)

        pltpu.sync_copy(
            gB.at[pl.ds(FSUB, FSUB)],
            out.at[c].at[pl.ds(tid * BPT + p * FSUB, FSUB)])


_gcn = functools.partial(
    pl.kernel,
    out_type=jax.ShapeDtypeStruct((NCORE, B2, HALF), jnp.float32),
    mesh=_MESH,
    compiler_params=_PARAMS,
    scratch_types=[
        pltpu.HBM((NCORE, N_PAD, HALF), jnp.float32),
        pltpu.HBM((NCORE, N_PAD, HALF), jnp.float32),
        pltpu.HBM((NCORE, N_PAD, HALF), jnp.float32),
        pltpu.VMEM((K, SUB), jnp.int32),
        pltpu.VMEM((K, SUB), jnp.int32),
        pltpu.VMEM((K, SUB), jnp.int32),
        pltpu.VMEM((K, SUB), jnp.int32),
        pltpu.VMEM((K, SUB), jnp.float32),
        pltpu.VMEM((K, SUB), jnp.float32),
        pltpu.VMEM((CHUNK, HALF), jnp.float32),
        pltpu.VMEM((CHUNK, HALF), jnp.float32),
        pltpu.VMEM_SHARED((N_PAD, HALF), jnp.float32),
        pltpu.SemaphoreType.DMA,
        pltpu.SemaphoreType.DMA,
        pltpu.SemaphoreType.DMA,
        pltpu.SemaphoreType.DMA,
        pltpu.SemaphoreType.DMA,
        pltpu.SemaphoreType.DMA,
        pltpu.SemaphoreType.DMA,
        pltpu.SemaphoreType.DMA,
        pltpu.SemaphoreType.DMA,
    ],
)(_gcn_body)


def kernel(users, items, user_emb, item_emb, adj_rows, adj_cols, adj_vals):
    ego0 = jnp.concatenate([user_emb, item_emb], axis=0)
    ego0 = jnp.pad(ego0, ((0, N_PAD - N), (0, 0)))
    ego0 = ego0.reshape(N_PAD, NCORE, HALF).transpose(1, 0, 2)

    cols3 = adj_cols[:E_MAIN].reshape(NCH, K, SUB)
    rows3 = adj_rows[:E_MAIN].reshape(NCH, K, SUB)
    vals3 = adj_vals[:E_MAIN].reshape(NCH, K, SUB)
    cols_t = adj_cols[E_MAIN:].reshape(TAIL // SUB, SUB)
    rows_t = adj_rows[E_MAIN:].reshape(TAIL // SUB, SUB)
    vals_t = adj_vals[E_MAIN:].reshape(TAIL // SUB, SUB)

    nid = jnp.concatenate(
        [users.astype(jnp.int32), items.astype(jnp.int32) + USERS])
    nid3 = nid.reshape(NSUB * 2, 2, SUB)

    outf = _gcn(ego0, cols3, rows3, vals3, cols_t, rows_t, vals_t, nid3)

    o = outf.transpose(1, 0, 2).reshape(B2, EMB)
    return (o[:BATCH], o[BATCH:])
